# Initial kernel scaffold; baseline (speedup 1.0000x reference)
import jax
import jax.numpy as jnp
from jax.experimental import pallas as pl


def kernel(x, edge_index, edge_weight, W1_0, W1_1, W1_2, b1, W2_0, W2_1, W2_2, b2, fc_w, fc_b):
    raise NotImplementedError("write your pallas kernel here")


# speedup vs baseline: 10.2390x; 10.2390x over previous
import functools

import jax
import jax.numpy as jnp
from jax import lax
from jax.experimental import pallas as pl
from jax.experimental.pallas import tpu as pltpu
from jax.experimental.pallas import tpu_sc as plsc

NC = 2
NS = 16
NW = NC * NS
L = 16

N = 10000
NP = 10240
E = 320000
CH = 128
EP = 327680
CPT = EP // (NW * CH)
RPT = NP // NS

F32 = jnp.float32


def _mesh():
    return plsc.VectorSubcoreMesh(core_axis_name="c", subcore_axis_name="s")


def _sc_deg(row2, wk2, zrow):
    @functools.partial(
        pl.kernel,
        mesh=_mesh(),
        out_type=jax.ShapeDtypeStruct((NC, NP), F32),
        scratch_types=[
            pltpu.VMEM((CPT, CH), jnp.int32),
            pltpu.VMEM((CPT, CH), F32),
            pltpu.VMEM_SHARED((NP,), F32),
        ],
    )
    def k(row_hbm, w_hbm, z_hbm, out_hbm, row_v, w_v, deg_s):
        c = lax.axis_index("c")
        s = lax.axis_index("s")
        base = (c * NS + s) * CPT
        pltpu.sync_copy(row_hbm.at[pl.ds(base, CPT)], row_v)
        pltpu.sync_copy(w_hbm.at[pl.ds(base, CPT)], w_v)
        pltpu.sync_copy(z_hbm.at[pl.ds(s * RPT, RPT)], deg_s.at[pl.ds(s * RPT, RPT)])
        plsc.subcore_barrier()

        @pl.loop(0, CPT)
        def _(j):
            pltpu.sync_copy(w_v.at[j], deg_s.at[row_v.at[j]], add=True)

        plsc.subcore_barrier()
        pltpu.sync_copy(deg_s.at[pl.ds(s * RPT, RPT)], out_hbm.at[c, pl.ds(s * RPT, RPT)])

    return k(row2, wk2, zrow)


def _sc_spmm(z, row2, col2, wk2, zslab):
    @functools.partial(
        pl.kernel,
        mesh=_mesh(),
        out_type=jax.ShapeDtypeStruct((NC, NP, 128), F32),
        scratch_types=[
            pltpu.VMEM((CPT, CH), jnp.int32),
            pltpu.VMEM((CPT, CH), jnp.int32),
            pltpu.VMEM((CPT, CH), F32),
            pltpu.VMEM((CH, 128), F32),
            pltpu.VMEM_SHARED((NP, 128), F32),
        ],
    )
    def k(z_hbm, row_hbm, col_hbm, w_hbm, zs_hbm, out_hbm,
          row_v, col_v, w_v, gbuf, acc_s):
        c = lax.axis_index("c")
        s = lax.axis_index("s")
        base = (c * NS + s) * CPT
        pltpu.sync_copy(row_hbm.at[pl.ds(base, CPT)], row_v)
        pltpu.sync_copy(col_hbm.at[pl.ds(base, CPT)], col_v)
        pltpu.sync_copy(w_hbm.at[pl.ds(base, CPT)], w_v)
        pltpu.sync_copy(zs_hbm, acc_s.at[pl.ds(s * RPT, RPT)])
        plsc.subcore_barrier()

        @pl.loop(0, CPT)
        def _(j):
            pltpu.sync_copy(z_hbm.at[row_v.at[j]], gbuf)

            dn = lax.GatherDimensionNumbers(
                offset_dims=(), collapsed_slice_dims=(0,), start_index_map=(0,))

            @pl.loop(0, CH // L)
            def _(g):
                wv = w_v[j, pl.ds(g * L, L)]
                for t in range(L):
                    sv = lax.gather(
                        wv, jnp.full((L, 1), t, jnp.int32), dn,
                        slice_sizes=(1,),
                        mode=lax.GatherScatterMode.PROMISE_IN_BOUNDS)
                    e = g * L + t
                    for f in range(128 // L):
                        sl = (e, pl.ds(f * L, L))
                        gbuf[sl] = gbuf[sl] * sv

            pltpu.sync_copy(gbuf, acc_s.at[col_v.at[j]], add=True)

        plsc.subcore_barrier()
        pltpu.sync_copy(acc_s.at[pl.ds(s * RPT, RPT)],
                        out_hbm.at[c, pl.ds(s * RPT, RPT)])

    return k(z, row2, col2, wk2, zslab)


BLK = 1024


def _tc_mask(row2, col2, w2):
    def body(r_ref, c_ref, w_ref, o_ref):
        o_ref[...] = jnp.where(r_ref[...] != c_ref[...], w_ref[...], 0.0)

    return pl.pallas_call(
        body, out_shape=jax.ShapeDtypeStruct((EP // CH, CH), F32))(row2, col2, w2)


def _tc_dis(dparts):
    def body(d_ref, o_ref):
        d = d_ref[0] + d_ref[1]
        safe = jnp.where(d > 0, d, 1.0)
        o_ref[...] = jnp.where(d > 0, lax.rsqrt(safe), 0.0)

    return pl.pallas_call(
        body, out_shape=jax.ShapeDtypeStruct((NP // 128, 128), F32))(dparts)


def _tc_scale(x, dcol):
    def body(x_ref, d_ref, o_ref):
        o_ref[...] = x_ref[...] * d_ref[...]

    return pl.pallas_call(
        body,
        grid=(NP // BLK,),
        in_specs=[
            pl.BlockSpec((BLK, 128), lambda i: (i, 0)),
            pl.BlockSpec((BLK, 1), lambda i: (i, 0)),
        ],
        out_specs=pl.BlockSpec((BLK, 128), lambda i: (i, 0)),
        out_shape=jax.ShapeDtypeStruct((NP, 128), F32),
    )(x, dcol)


def _tc_comb(p, dcol):
    def body(p_ref, d_ref, tx_ref, u_ref):
        d = d_ref[...]
        t = (p_ref[0] + p_ref[1]) * d
        tx_ref[...] = -t
        u_ref[...] = -t * d

    return pl.pallas_call(
        body,
        grid=(NP // BLK,),
        in_specs=[
            pl.BlockSpec((2, BLK, 128), lambda i: (0, i, 0)),
            pl.BlockSpec((BLK, 1), lambda i: (i, 0)),
        ],
        out_specs=[
            pl.BlockSpec((BLK, 128), lambda i: (i, 0)),
            pl.BlockSpec((BLK, 128), lambda i: (i, 0)),
        ],
        out_shape=[
            jax.ShapeDtypeStruct((NP, 128), F32),
            jax.ShapeDtypeStruct((NP, 128), F32),
        ],
    )(p, dcol)


def _tc_layer(x, tx1, q, dcol, w0, w1, w2, b):
    def body(x_ref, t1_ref, q_ref, d_ref, w0_ref, w1_ref, w2_ref, b_ref,
             h_ref, u_ref):
        xb = x_ref[...]
        d = d_ref[...]
        t1 = t1_ref[...]
        t2 = -2.0 * d * (q_ref[0] + q_ref[1]) - xb
        acc = jnp.dot(xb, w0_ref[...], preferred_element_type=F32)
        acc += jnp.dot(t1, w1_ref[...], preferred_element_type=F32)
        acc += jnp.dot(t2, w2_ref[...], preferred_element_type=F32)
        acc += b_ref[...]
        h = jnp.maximum(acc, 0.0)
        h_ref[...] = h
        u_ref[...] = h * d

    wspec = pl.BlockSpec((128, 128), lambda i: (0, 0))
    return pl.pallas_call(
        body,
        grid=(NP // BLK,),
        in_specs=[
            pl.BlockSpec((BLK, 128), lambda i: (i, 0)),
            pl.BlockSpec((BLK, 128), lambda i: (i, 0)),
            pl.BlockSpec((2, BLK, 128), lambda i: (0, i, 0)),
            pl.BlockSpec((BLK, 1), lambda i: (i, 0)),
            wspec, wspec, wspec,
            pl.BlockSpec((1, 128), lambda i: (0, 0)),
        ],
        out_specs=[
            pl.BlockSpec((BLK, 128), lambda i: (i, 0)),
            pl.BlockSpec((BLK, 128), lambda i: (i, 0)),
        ],
        out_shape=[
            jax.ShapeDtypeStruct((NP, 128), F32),
            jax.ShapeDtypeStruct((NP, 128), F32),
        ],
    )(x, tx1, q, dcol, w0, w1, w2, b)


def _tc_layer_fc(x, tx1, q, dcol, w0, w1, w2, b, fcw, fcb):
    def body(x_ref, t1_ref, q_ref, d_ref, w0_ref, w1_ref, w2_ref, b_ref,
             fw_ref, fb_ref, o_ref):
        xb = x_ref[...]
        t1 = t1_ref[...]
        t2 = -2.0 * d_ref[...] * (q_ref[0] + q_ref[1]) - xb
        acc = jnp.dot(xb, w0_ref[...], preferred_element_type=F32)
        acc += jnp.dot(t1, w1_ref[...], preferred_element_type=F32)
        acc += jnp.dot(t2, w2_ref[...], preferred_element_type=F32)
        acc += b_ref[...]
        h = jnp.maximum(acc, 0.0)
        o_ref[...] = jnp.dot(h, fw_ref[...], preferred_element_type=F32) + fb_ref[...]

    wspec = pl.BlockSpec((128, 128), lambda i: (0, 0))
    bspec = pl.BlockSpec((1, 128), lambda i: (0, 0))
    return pl.pallas_call(
        body,
        grid=(NP // BLK,),
        in_specs=[
            pl.BlockSpec((BLK, 128), lambda i: (i, 0)),
            pl.BlockSpec((BLK, 128), lambda i: (i, 0)),
            pl.BlockSpec((2, BLK, 128), lambda i: (0, i, 0)),
            pl.BlockSpec((BLK, 1), lambda i: (i, 0)),
            wspec, wspec, wspec, bspec,
            wspec, bspec,
        ],
        out_specs=pl.BlockSpec((BLK, 128), lambda i: (i, 0)),
        out_shape=jax.ShapeDtypeStruct((NP, 128), F32),
    )(x, tx1, q, dcol, w0, w1, w2, b, fcw, fcb)


def kernel(x, edge_index, edge_weight, W1_0, W1_1, W1_2, b1,
           W2_0, W2_1, W2_2, b2, fc_w, fc_b):
    row = edge_index[0]
    col = edge_index[1]
    pad = EP - E
    pidx = (jnp.arange(pad, dtype=jnp.int32) % NP).astype(jnp.int32)
    row2 = jnp.concatenate([row, pidx]).reshape(EP // CH, CH)
    col2 = jnp.concatenate([col, pidx]).reshape(EP // CH, CH)
    w2e = jnp.concatenate([edge_weight, jnp.zeros((pad,), F32)]).reshape(EP // CH, CH)

    xp = jnp.zeros((NP, 128), F32).at[:N].set(x)
    zrow = jnp.zeros((NP,), F32)
    zslab = jnp.zeros((RPT, 128), F32)

    wk2 = _tc_mask(row2, col2, w2e)
    dparts = _sc_deg(row2, wk2, zrow)
    dis = _tc_dis(dparts.reshape(NC, NP // 128, 128))
    dcol = dis.reshape(NP, 1)

    b1r = b1.reshape(1, 128)
    b2r = b2.reshape(1, 128)
    fcw = jnp.zeros((128, 128), F32).at[:, :2].set(fc_w)
    fcb = jnp.zeros((1, 128), F32).at[0, :2].set(fc_b)

    u0 = _tc_scale(xp, dcol)
    p = _sc_spmm(u0, row2, col2, wk2, zslab)
    tx1, u1 = _tc_comb(p, dcol)
    q = _sc_spmm(u1, row2, col2, wk2, zslab)
    h1, uh = _tc_layer(xp, tx1, q, dcol, W1_0, W1_1, W1_2, b1r)

    p2 = _sc_spmm(uh, row2, col2, wk2, zslab)
    tu1, u2 = _tc_comb(p2, dcol)
    q2 = _sc_spmm(u2, row2, col2, wk2, zslab)
    out = _tc_layer_fc(h1, tu1, q2, dcol, W2_0, W2_1, W2_2, b2r, fcw, fcb)

    return out[:N, :2]


# speedup vs baseline: 15.6090x; 1.5245x over previous
import functools

import jax
import jax.numpy as jnp
from jax import lax
from jax.experimental import pallas as pl
from jax.experimental.pallas import tpu as pltpu
from jax.experimental.pallas import tpu_sc as plsc

NC = 2
NS = 16
NW = NC * NS
L = 16

N = 10000
NP = 10240
E = 320000
CH = 128
EP = 327680
CPT = EP // (NW * CH)
RPT = NP // NS

F32 = jnp.float32


def _mesh():
    return plsc.VectorSubcoreMesh(core_axis_name="c", subcore_axis_name="s")


def _sc_deg(row2, wk2, zrow):
    @functools.partial(
        pl.kernel,
        mesh=_mesh(),
        out_type=jax.ShapeDtypeStruct((NC, NP), F32),
        scratch_types=[
            pltpu.VMEM((CPT, CH), jnp.int32),
            pltpu.VMEM((CPT, CH), F32),
            pltpu.VMEM_SHARED((NP,), F32),
        ],
    )
    def k(row_hbm, w_hbm, z_hbm, out_hbm, row_v, w_v, deg_s):
        c = lax.axis_index("c")
        s = lax.axis_index("s")
        base = (c * NS + s) * CPT
        pltpu.sync_copy(row_hbm.at[pl.ds(base, CPT)], row_v)
        pltpu.sync_copy(w_hbm.at[pl.ds(base, CPT)], w_v)
        pltpu.sync_copy(z_hbm.at[pl.ds(s * RPT, RPT)], deg_s.at[pl.ds(s * RPT, RPT)])
        plsc.subcore_barrier()

        @pl.loop(0, CPT)
        def _(j):
            pltpu.sync_copy(w_v.at[j], deg_s.at[row_v.at[j]], add=True)

        plsc.subcore_barrier()
        pltpu.sync_copy(deg_s.at[pl.ds(s * RPT, RPT)], out_hbm.at[c, pl.ds(s * RPT, RPT)])

    return k(row2, wk2, zrow)


def _sc_spmm(z, row2, col2, wk2, zslab):
    @functools.partial(
        pl.kernel,
        mesh=_mesh(),
        out_type=jax.ShapeDtypeStruct((NC, NP, 128), F32),
        scratch_types=[
            pltpu.VMEM((CPT // 2, CH), jnp.int32),
            pltpu.VMEM((CPT // 2, CH), jnp.int32),
            pltpu.VMEM((CPT // 2, CH), F32),
            pltpu.VMEM((CH, 128), F32),
            pltpu.VMEM((CH, 128), F32),
            pltpu.VMEM_SHARED((NP, 128), F32),
            pltpu.SemaphoreType.DMA,
            pltpu.SemaphoreType.DMA,
        ],
    )
    def k(z_hbm, row_hbm, col_hbm, w_hbm, zs_hbm, out_hbm,
          row_v, col_v, w_v, gbuf0, gbuf1, acc_s, sem0, sem1):
        c = lax.axis_index("c")
        s = lax.axis_index("s")
        CPH = CPT // 2
        base = (c * NS + s) * CPT
        pltpu.sync_copy(zs_hbm, acc_s.at[pl.ds(s * RPT, RPT)])
        plsc.subcore_barrier()

        dn = lax.GatherDimensionNumbers(
            offset_dims=(), collapsed_slice_dims=(0,), start_index_map=(0,))

        def scale_scatter(gbuf, j):
            @pl.loop(0, CH // L)
            def _(g):
                wv = w_v[j, pl.ds(g * L, L)]
                for t in range(L):
                    sv = lax.gather(
                        wv, jnp.full((L, 1), t, jnp.int32), dn,
                        slice_sizes=(1,),
                        mode=lax.GatherScatterMode.PROMISE_IN_BOUNDS)
                    e = g * L + t
                    for f in range(128 // L):
                        sl = (e, pl.ds(f * L, L))
                        gbuf[sl] = gbuf[sl] * sv

            pltpu.sync_copy(gbuf, acc_s.at[col_v.at[j]], add=True)

        for h in range(2):
            bh = base + h * CPH
            pltpu.sync_copy(row_hbm.at[pl.ds(bh, CPH)], row_v)
            pltpu.sync_copy(col_hbm.at[pl.ds(bh, CPH)], col_v)
            pltpu.sync_copy(w_hbm.at[pl.ds(bh, CPH)], w_v)

            pltpu.make_async_copy(z_hbm.at[row_v.at[0]], gbuf0, sem0).start()

            @pl.loop(0, CPH // 2)
            def _(p):
                j0 = 2 * p
                j1 = j0 + 1
                pltpu.make_async_copy(
                    z_hbm.at[row_v.at[j1]], gbuf1, sem1).start()
                pltpu.make_async_copy(
                    z_hbm.at[row_v.at[j0]], gbuf0, sem0).wait()
                scale_scatter(gbuf0, j0)

                @pl.when(j1 + 1 < CPH)
                def _():
                    pltpu.make_async_copy(
                        z_hbm.at[row_v.at[j1 + 1]], gbuf0, sem0).start()

                pltpu.make_async_copy(
                    z_hbm.at[row_v.at[j1]], gbuf1, sem1).wait()
                scale_scatter(gbuf1, j1)

        plsc.subcore_barrier()
        pltpu.sync_copy(acc_s.at[pl.ds(s * RPT, RPT)],
                        out_hbm.at[c, pl.ds(s * RPT, RPT)])

    return k(z, row2, col2, wk2, zslab)


BLK = 1024


def _tc_mask(row2, col2, w2):
    def body(r_ref, c_ref, w_ref, o_ref):
        o_ref[...] = jnp.where(r_ref[...] != c_ref[...], w_ref[...], 0.0)

    return pl.pallas_call(
        body, out_shape=jax.ShapeDtypeStruct((EP // CH, CH), F32))(row2, col2, w2)


def _tc_dis(dparts):
    def body(d_ref, o_ref):
        d = d_ref[0] + d_ref[1]
        safe = jnp.where(d > 0, d, 1.0)
        o_ref[...] = jnp.where(d > 0, lax.rsqrt(safe), 0.0)

    return pl.pallas_call(
        body, out_shape=jax.ShapeDtypeStruct((NP // 128, 128), F32))(dparts)


def _tc_scale(x, dcol):
    def body(x_ref, d_ref, o_ref):
        o_ref[...] = x_ref[...] * d_ref[...]

    return pl.pallas_call(
        body,
        grid=(NP // BLK,),
        in_specs=[
            pl.BlockSpec((BLK, 128), lambda i: (i, 0)),
            pl.BlockSpec((BLK, 1), lambda i: (i, 0)),
        ],
        out_specs=pl.BlockSpec((BLK, 128), lambda i: (i, 0)),
        out_shape=jax.ShapeDtypeStruct((NP, 128), F32),
    )(x, dcol)


def _tc_comb(p, dcol):
    def body(p_ref, d_ref, tx_ref, u_ref):
        d = d_ref[...]
        t = (p_ref[0] + p_ref[1]) * d
        tx_ref[...] = -t
        u_ref[...] = -t * d

    return pl.pallas_call(
        body,
        grid=(NP // BLK,),
        in_specs=[
            pl.BlockSpec((2, BLK, 128), lambda i: (0, i, 0)),
            pl.BlockSpec((BLK, 1), lambda i: (i, 0)),
        ],
        out_specs=[
            pl.BlockSpec((BLK, 128), lambda i: (i, 0)),
            pl.BlockSpec((BLK, 128), lambda i: (i, 0)),
        ],
        out_shape=[
            jax.ShapeDtypeStruct((NP, 128), F32),
            jax.ShapeDtypeStruct((NP, 128), F32),
        ],
    )(p, dcol)


def _tc_layer(x, tx1, q, dcol, w0, w1, w2, b):
    def body(x_ref, t1_ref, q_ref, d_ref, w0_ref, w1_ref, w2_ref, b_ref,
             h_ref, u_ref):
        xb = x_ref[...]
        d = d_ref[...]
        t1 = t1_ref[...]
        t2 = -2.0 * d * (q_ref[0] + q_ref[1]) - xb
        acc = jnp.dot(xb, w0_ref[...], preferred_element_type=F32)
        acc += jnp.dot(t1, w1_ref[...], preferred_element_type=F32)
        acc += jnp.dot(t2, w2_ref[...], preferred_element_type=F32)
        acc += b_ref[...]
        h = jnp.maximum(acc, 0.0)
        h_ref[...] = h
        u_ref[...] = h * d

    wspec = pl.BlockSpec((128, 128), lambda i: (0, 0))
    return pl.pallas_call(
        body,
        grid=(NP // BLK,),
        in_specs=[
            pl.BlockSpec((BLK, 128), lambda i: (i, 0)),
            pl.BlockSpec((BLK, 128), lambda i: (i, 0)),
            pl.BlockSpec((2, BLK, 128), lambda i: (0, i, 0)),
            pl.BlockSpec((BLK, 1), lambda i: (i, 0)),
            wspec, wspec, wspec,
            pl.BlockSpec((1, 128), lambda i: (0, 0)),
        ],
        out_specs=[
            pl.BlockSpec((BLK, 128), lambda i: (i, 0)),
            pl.BlockSpec((BLK, 128), lambda i: (i, 0)),
        ],
        out_shape=[
            jax.ShapeDtypeStruct((NP, 128), F32),
            jax.ShapeDtypeStruct((NP, 128), F32),
        ],
    )(x, tx1, q, dcol, w0, w1, w2, b)


def _tc_layer_fc(x, tx1, q, dcol, w0, w1, w2, b, fcw, fcb):
    def body(x_ref, t1_ref, q_ref, d_ref, w0_ref, w1_ref, w2_ref, b_ref,
             fw_ref, fb_ref, o_ref):
        xb = x_ref[...]
        t1 = t1_ref[...]
        t2 = -2.0 * d_ref[...] * (q_ref[0] + q_ref[1]) - xb
        acc = jnp.dot(xb, w0_ref[...], preferred_element_type=F32)
        acc += jnp.dot(t1, w1_ref[...], preferred_element_type=F32)
        acc += jnp.dot(t2, w2_ref[...], preferred_element_type=F32)
        acc += b_ref[...]
        h = jnp.maximum(acc, 0.0)
        o_ref[...] = jnp.dot(h, fw_ref[...], preferred_element_type=F32) + fb_ref[...]

    wspec = pl.BlockSpec((128, 128), lambda i: (0, 0))
    bspec = pl.BlockSpec((1, 128), lambda i: (0, 0))
    return pl.pallas_call(
        body,
        grid=(NP // BLK,),
        in_specs=[
            pl.BlockSpec((BLK, 128), lambda i: (i, 0)),
            pl.BlockSpec((BLK, 128), lambda i: (i, 0)),
            pl.BlockSpec((2, BLK, 128), lambda i: (0, i, 0)),
            pl.BlockSpec((BLK, 1), lambda i: (i, 0)),
            wspec, wspec, wspec, bspec,
            wspec, bspec,
        ],
        out_specs=pl.BlockSpec((BLK, 128), lambda i: (i, 0)),
        out_shape=jax.ShapeDtypeStruct((NP, 128), F32),
    )(x, tx1, q, dcol, w0, w1, w2, b, fcw, fcb)


def kernel(x, edge_index, edge_weight, W1_0, W1_1, W1_2, b1,
           W2_0, W2_1, W2_2, b2, fc_w, fc_b):
    row = edge_index[0]
    col = edge_index[1]
    pad = EP - E
    pidx = (jnp.arange(pad, dtype=jnp.int32) % NP).astype(jnp.int32)
    row2 = jnp.concatenate([row, pidx]).reshape(EP // CH, CH)
    col2 = jnp.concatenate([col, pidx]).reshape(EP // CH, CH)
    w2e = jnp.concatenate([edge_weight, jnp.zeros((pad,), F32)]).reshape(EP // CH, CH)

    xp = jnp.zeros((NP, 128), F32).at[:N].set(x)
    zrow = jnp.zeros((NP,), F32)
    zslab = jnp.zeros((RPT, 128), F32)

    wk2 = _tc_mask(row2, col2, w2e)
    dparts = _sc_deg(row2, wk2, zrow)
    dis = _tc_dis(dparts.reshape(NC, NP // 128, 128))
    dcol = dis.reshape(NP, 1)

    b1r = b1.reshape(1, 128)
    b2r = b2.reshape(1, 128)
    fcw = jnp.zeros((128, 128), F32).at[:, :2].set(fc_w)
    fcb = jnp.zeros((1, 128), F32).at[0, :2].set(fc_b)

    u0 = _tc_scale(xp, dcol)
    p = _sc_spmm(u0, row2, col2, wk2, zslab)
    tx1, u1 = _tc_comb(p, dcol)
    q = _sc_spmm(u1, row2, col2, wk2, zslab)
    h1, uh = _tc_layer(xp, tx1, q, dcol, W1_0, W1_1, W1_2, b1r)

    p2 = _sc_spmm(uh, row2, col2, wk2, zslab)
    tu1, u2 = _tc_comb(p2, dcol)
    q2 = _sc_spmm(u2, row2, col2, wk2, zslab)
    out = _tc_layer_fc(h1, tu1, q2, dcol, W2_0, W2_1, W2_2, b2r, fcw, fcb)

    return out[:N, :2]
